# baseline (speedup 1.0000x reference)
import functools
import jax
import jax.numpy as jnp
from jax import lax
from jax.experimental import pallas as pl
from jax.experimental.pallas import tpu as pltpu, tpu_sc as plsc

N_FLAT = 200 * 256
LANES = 16


def _sc_body(ew_hbm, pos_hbm, out_hbm, base_v, pos_v, sem):
    nc = 2
    wid = lax.axis_index("s") * nc + lax.axis_index("c")
    pltpu.sync_copy(ew_hbm, base_v)
    pltpu.sync_copy(pos_hbm, pos_v)

    def add_body(i, carry):
        sl = pl.ds(i * LANES, LANES)
        base_v[sl] = base_v[sl] + pos_v[sl]
        return carry

    lax.fori_loop(0, N_FLAT // LANES, add_body, 0)

    bpw = 1024 // 32
    copies = []
    for j in range(bpw):
        b = wid * bpw + j
        copies.append(
            pltpu.async_copy(base_v, out_hbm.at[pl.ds(b * N_FLAT, N_FLAT)], sem)
        )
    for c in copies:
        c.wait()


def kernel(x, embed_weight, pos):
    b = x.shape[0]
    n, d = embed_weight.shape
    mesh = plsc.VectorSubcoreMesh(core_axis_name="c", subcore_axis_name="s")
    k = functools.partial(
        pl.kernel,
        mesh=mesh,
        out_type=jax.ShapeDtypeStruct((b * n * d,), jnp.float32),
        scratch_types=[
            pltpu.VMEM((n * d,), jnp.float32),
            pltpu.VMEM((n * d,), jnp.float32),
            pltpu.SemaphoreType.DMA,
        ],
    )(_sc_body)
    out = k(embed_weight.reshape(-1), pos.reshape(-1))
    return out.reshape(b, n, d)


# speedup vs baseline: 5.1709x; 5.1709x over previous
import jax
import jax.numpy as jnp
from jax.experimental import pallas as pl
from jax.experimental.pallas import tpu as pltpu

REP = 64
CHUNK = 8


def _body(ew_ref, pos_ref, out_ref, scratch, sem):
    base = ew_ref[...] + pos_ref[...]
    b = out_ref.shape[0]
    copies = []
    for c in range(REP // CHUNK):
        for r in range(c * CHUNK, (c + 1) * CHUNK):
            scratch[r] = base
        copies.append(
            pltpu.make_async_copy(
                scratch.at[pl.ds(c * CHUNK, CHUNK)],
                out_ref.at[pl.ds(c * CHUNK, CHUNK)],
                sem,
            )
        )
        copies[-1].start()
    for i in range(1, b // REP):
        copies.append(
            pltpu.make_async_copy(scratch, out_ref.at[pl.ds(i * REP, REP)], sem)
        )
        copies[-1].start()
    for c in copies:
        c.wait()


def kernel(x, embed_weight, pos):
    b = x.shape[0]
    n, d = embed_weight.shape
    return pl.pallas_call(
        _body,
        in_specs=[
            pl.BlockSpec(memory_space=pltpu.VMEM),
            pl.BlockSpec(memory_space=pltpu.VMEM),
        ],
        out_specs=pl.BlockSpec(memory_space=pl.ANY),
        out_shape=jax.ShapeDtypeStruct((b, n, d), jnp.float32),
        scratch_shapes=[
            pltpu.VMEM((REP, n, d), jnp.float32),
            pltpu.SemaphoreType.DMA,
        ],
    )(embed_weight, pos)
